# Initial kernel scaffold; baseline (speedup 1.0000x reference)
import jax
import jax.numpy as jnp
from jax.experimental import pallas as pl


def kernel(anchors, objectness, box_regression):
    raise NotImplementedError("write your pallas kernel here")


# speedup vs baseline: 5.7513x; 5.7513x over previous
import math

import jax
import jax.numpy as jnp
from jax.experimental import pallas as pl
from jax.experimental.pallas import tpu as pltpu

_PRE_N = 2000
_PAD_N = 2048
_POST_N = 1000
_BLK = 128
_NBLK = _PAD_N // _BLK
_TH = 0.7
_IMG = 800.0
_CLIP = float(math.log(1000.0 / 16.0))


def _rpn_kernel(logits_ref, reg_ref, anc_ref, scores_ref, props_ref, keep_ref,
                s_in, s_kb):
    l = logits_ref[0]
    scores_ref[...] = jax.nn.sigmoid(l)[None]

    ax1 = anc_ref[:, 0, :]
    ay1 = anc_ref[:, 1, :]
    ax2 = anc_ref[:, 2, :]
    ay2 = anc_ref[:, 3, :]
    dx = reg_ref[:, 0, :]
    dy = reg_ref[:, 1, :]
    dw = jnp.minimum(reg_ref[:, 2, :], _CLIP)
    dh = jnp.minimum(reg_ref[:, 3, :], _CLIP)

    widths = ax2 - ax1 + 1.0
    heights = ay2 - ay1 + 1.0
    ctr_x = ax1 + 0.5 * widths
    ctr_y = ay1 + 0.5 * heights
    pred_ctr_x = dx * widths + ctr_x
    pred_ctr_y = dy * heights + ctr_y
    pred_w = jnp.exp(dw) * widths
    pred_h = jnp.exp(dh) * heights

    x1 = jnp.clip(pred_ctr_x - 0.5 * pred_w, 0.0, _IMG - 1.0)
    y1 = jnp.clip(pred_ctr_y - 0.5 * pred_h, 0.0, _IMG - 1.0)
    x2 = jnp.clip(pred_ctr_x + 0.5 * pred_w - 1.0, 0.0, _IMG - 1.0)
    y2 = jnp.clip(pred_ctr_y + 0.5 * pred_h - 1.0, 0.0, _IMG - 1.0)

    props_ref[...] = jnp.concatenate([x1, y1, x2, y2], axis=0)[None]

    ws = x2 - x1 + 1.0
    hs = y2 - y1 + 1.0
    x_ctr = x1 + ws / 2.0
    y_ctr = y1 + hs / 2.0

    col = jax.lax.broadcasted_iota(jnp.int32, (1, _PAD_N), 1)
    valid = (ws >= 0.0) & (hs >= 0.0) & (x_ctr < _IMG) & (y_ctr < _IMG)
    keep = (valid & (col < _PRE_N)).astype(jnp.float32)

    areas = ws * hs
    x1t = jnp.reshape(x1, (_PAD_N, 1))
    y1t = jnp.reshape(y1, (_PAD_N, 1))
    x2t = jnp.reshape(x2, (_PAD_N, 1))
    y2t = jnp.reshape(y2, (_PAD_N, 1))
    art = jnp.reshape(areas, (_PAD_N, 1))

    lcol = jax.lax.broadcasted_iota(jnp.int32, (1, _BLK), 1)

    for b in range(_NBLK):
        lo, hi = b * _BLK, (b + 1) * _BLK
        xx1 = jnp.maximum(x1t[lo:hi], x1)
        yy1 = jnp.maximum(y1t[lo:hi], y1)
        xx2 = jnp.minimum(x2t[lo:hi], x2)
        yy2 = jnp.minimum(y2t[lo:hi], y2)
        w = jnp.maximum(xx2 - xx1 + 1.0, 0.0)
        h = jnp.maximum(yy2 - yy1 + 1.0, 0.0)
        inter = w * h
        iou_blk = inter / (art[lo:hi] + areas - inter)
        over = (iou_blk > _TH).astype(jnp.float32)

        s_in[...] = over[:, lo:hi]
        s_kb[...] = keep[:, lo:hi]

        def body(i, carry):
            row = s_in[pl.ds(i, 1), :]
            kv = s_kb[...]
            ki = jnp.max(kv * (lcol == i).astype(jnp.float32))
            sup = row * (lcol > i).astype(jnp.float32) * ki
            s_kb[...] = s_kb[...] * (1.0 - sup)
            return carry

        jax.lax.fori_loop(0, _BLK, body, 0)
        kb = s_kb[...]
        parts = ([keep[:, :lo]] if lo > 0 else []) + [kb] \
            + ([keep[:, hi:]] if hi < _PAD_N else [])
        keep = jnp.concatenate(parts, axis=1) if len(parts) > 1 else kb
        if b + 1 < _NBLK:
            kbt = jnp.reshape(kb, (_BLK, 1))
            sup_any = jnp.max(over * kbt, axis=0, keepdims=True)
            later = (col >= hi).astype(jnp.float32)
            keep = keep * (1.0 - sup_any * later)

    keep_ref[...] = (keep > 0.5).astype(jnp.int32)[None]


@jax.jit
def kernel(anchors, objectness, box_regression):
    N, A, H, W = objectness.shape
    obj = jnp.transpose(objectness, (0, 2, 3, 1)).reshape(N, -1)
    reg = box_regression.reshape(N, A, 4, H, W)
    reg = jnp.transpose(reg, (0, 3, 4, 1, 2)).reshape(N, -1, 4)

    top_logits, top_idx = jax.lax.top_k(obj, _PRE_N)
    reg_t = jnp.take_along_axis(reg, top_idx[..., None], axis=1)
    anc_t = jnp.take_along_axis(anchors, top_idx[..., None], axis=1)

    pad = _PAD_N - _PRE_N
    logits_p = jnp.pad(top_logits, ((0, 0), (0, pad)),
                       constant_values=-30.0)[:, None, :]
    reg_p = jnp.transpose(jnp.pad(reg_t, ((0, 0), (0, pad), (0, 0))), (0, 2, 1))
    anc_p = jnp.transpose(jnp.pad(anc_t, ((0, 0), (0, pad), (0, 0))), (0, 2, 1))

    scores, props, keep = pl.pallas_call(
        _rpn_kernel,
        grid=(N,),
        in_specs=[
            pl.BlockSpec((1, 1, _PAD_N), lambda i: (i, 0, 0)),
            pl.BlockSpec((1, 4, _PAD_N), lambda i: (i, 0, 0)),
            pl.BlockSpec((1, 4, _PAD_N), lambda i: (i, 0, 0)),
        ],
        out_specs=[
            pl.BlockSpec((1, 1, _PAD_N), lambda i: (i, 0, 0)),
            pl.BlockSpec((1, 4, _PAD_N), lambda i: (i, 0, 0)),
            pl.BlockSpec((1, 1, _PAD_N), lambda i: (i, 0, 0)),
        ],
        out_shape=[
            jax.ShapeDtypeStruct((N, 1, _PAD_N), jnp.float32),
            jax.ShapeDtypeStruct((N, 4, _PAD_N), jnp.float32),
            jax.ShapeDtypeStruct((N, 1, _PAD_N), jnp.int32),
        ],
        scratch_shapes=[
            pltpu.VMEM((_BLK, _BLK), jnp.float32),
            pltpu.VMEM((1, _BLK), jnp.float32),
        ],
    )(logits_p, reg_p, anc_p)

    keep2 = keep[:, 0, :_PRE_N] > 0
    props2 = jnp.transpose(props, (0, 2, 1))[:, :_PRE_N, :]
    scores2 = scores[:, 0, :_PRE_N]

    ar = jnp.arange(_PRE_N)
    order = jnp.where(keep2, ar[None, :], _PRE_N + ar[None, :])
    sel = jnp.argsort(order, axis=1)[:, :_POST_N]
    kept = jnp.take_along_axis(keep2, sel, axis=1)
    out_boxes = jnp.where(kept[..., None],
                          jnp.take_along_axis(props2, sel[..., None], axis=1),
                          0.0)
    out_scores = jnp.where(kept, jnp.take_along_axis(scores2, sel, axis=1), -1.0)
    return jnp.concatenate([out_boxes, out_scores[..., None]], axis=2)
